# baseline (speedup 1.0000x reference)
import functools

import jax
import jax.numpy as jnp
from jax import lax
from jax.experimental import pallas as pl
from jax.experimental.pallas import tpu as pltpu
from jax.experimental.pallas import tpu_sc as plsc

D = 2048

_info = plsc.get_sparse_core_info()
NC, NS, L = _info.num_cores, _info.num_subcores, _info.num_lanes
NW = NC * NS

B = 4 * 4096
B_PER_W = B // NW
CH = 16
N_CHUNKS = B_PER_W // CH
NBUF = 3
N_TRI = N_CHUNKS // NBUF
N_TAIL = N_CHUNKS - NBUF * N_TRI


def _make_gather():
    mesh = plsc.VectorSubcoreMesh(core_axis_name="c", subcore_axis_name="s")

    @functools.partial(
        pl.kernel,
        mesh=mesh,
        out_type=jax.ShapeDtypeStruct((B, D), jnp.float32),
        scratch_types=[
            pltpu.VMEM((N_CHUNKS, CH), jnp.int32),
            pltpu.VMEM((CH, D), jnp.float32),
            pltpu.VMEM((CH, D), jnp.float32),
            pltpu.VMEM((CH, D), jnp.float32),
            pltpu.SemaphoreType.DMA,
            pltpu.SemaphoreType.DMA,
            pltpu.SemaphoreType.DMA,
            pltpu.SemaphoreType.DMA,
            pltpu.SemaphoreType.DMA,
            pltpu.SemaphoreType.DMA,
        ],
    )
    def k(table_hbm, idx_hbm, out_hbm, idx_v,
          buf0, buf1, buf2, g0, g1, g2, w0, w1, w2):
        wid = lax.axis_index("s") * NC + lax.axis_index("c")
        base = wid * B_PER_W
        pltpu.sync_copy(idx_hbm.at[wid], idx_v)

        bufs = ((buf0, g0, w0), (buf1, g1, w1), (buf2, g2, w2))

        def fire_gather(c, buf, sem):
            pltpu.async_copy(table_hbm.at[idx_v.at[c]], buf, sem)

        def wait_gather(c, buf, sem):
            pltpu.make_async_copy(table_hbm.at[idx_v.at[c]], buf, sem).wait()

        def fire_write(c, buf, sem):
            pltpu.async_copy(buf, out_hbm.at[pl.ds(base + c * CH, CH)], sem)

        def wait_write(c, buf, sem):
            pltpu.make_async_copy(
                buf, out_hbm.at[pl.ds(base + c * CH, CH)], sem
            ).wait()

        for j in range(NBUF):
            fire_gather(j, bufs[j][0], bufs[j][1])

        def body(i, carry):
            c0 = NBUF * i
            for j in range(NBUF):
                c = c0 + j
                buf, g, w = bufs[j]
                wait_gather(c, buf, g)
                fire_write(c, buf, w)
            for j in range(NBUF):
                c = c0 + j
                buf, g, w = bufs[j]
                wait_write(c, buf, w)

                @pl.when(c + NBUF < N_CHUNKS)
                def _():
                    fire_gather(c + NBUF, buf, g)

            return carry

        lax.fori_loop(0, N_TRI, body, 0, unroll=False)

        for j in range(N_TAIL):
            c = NBUF * N_TRI + j
            buf, g, w = bufs[j]
            wait_gather(c, buf, g)
            fire_write(c, buf, w)
        for j in range(N_TAIL):
            c = NBUF * N_TRI + j
            buf, g, w = bufs[j]
            wait_write(c, buf, w)

    return k


_gather = _make_gather()


def kernel(x, weight):
    idx = x.reshape(NW, N_CHUNKS, CH).astype(jnp.int32)
    out = _gather(weight, idx)
    return out.reshape(x.shape + (D,))
